# baseline (speedup 1.0000x reference)
import jax
import jax.numpy as jnp
from jax import lax
from jax.experimental import pallas as pl
from jax.experimental.pallas import tpu as pltpu
from jax.experimental.pallas import tpu_sc as plsc

N_NODES = 10000
N_EDGES = 320000
D = 128
DE = 16
NEG = 0.01

NC = 2
NS = 16
NW = NC * NS

EPW = N_EDGES // NW
C = 80
NCH = EPW // C
NPAIR = (NCH - 1) // 2

N_PAD = 10240
ROWS_PER_TILE = N_PAD // NS

def _leaky(v):
    return jnp.where(v >= 0, v, NEG * v)


def _prep_body(x_ref, wt_ref, b_ref, h_ref, hp_ref):
    v = jnp.dot(x_ref[...], wt_ref[...], preferred_element_type=jnp.float32)
    h = _leaky(v + b_ref[...])
    h_ref[...] = h
    r = lax.bitcast_convert_type(h, jnp.uint32) + jnp.uint32(0x8000)
    hp_ref[...] = (r[:, :D // 2] >> jnp.uint32(16)) | (r[:, D // 2:] & jnp.uint32(0xFFFF0000))


def _prep(x, wt, b):
    rb = 2000
    return pl.pallas_call(
        _prep_body,
        grid=(N_NODES // rb,),
        in_specs=[
            pl.BlockSpec((rb, D), lambda i: (i, 0)),
            pl.BlockSpec((D, D), lambda i: (0, 0)),
            pl.BlockSpec((1, D), lambda i: (0, 0)),
        ],
        out_specs=[pl.BlockSpec((rb, D), lambda i: (i, 0)),
                   pl.BlockSpec((rb, D // 2), lambda i: (i, 0))],
        out_shape=[jax.ShapeDtypeStruct((N_NODES, D), jnp.float32),
                   jax.ShapeDtypeStruct((N_NODES, D // 2), jnp.uint32)],
    )(x, wt, b)


def _edge_body(ewt_ref, w_ref, b_ref, e_ref):
    v = lax.dot_general(ewt_ref[...], w_ref[...],
                        dimension_numbers=(((0,), (0,)), ((), ())),
                        preferred_element_type=jnp.float32)
    v = v + b_ref[...]
    r = lax.bitcast_convert_type(v, jnp.uint32) + jnp.uint32(0x8000)
    n2 = r.shape[0] // 2
    p1 = (r[:n2, :D // 2] >> jnp.uint32(16)) | (r[:n2, D // 2:] & jnp.uint32(0xFFFF0000))
    p2 = (r[n2:, :D // 2] >> jnp.uint32(16)) | (r[n2:, D // 2:] & jnp.uint32(0xFFFF0000))
    e_ref[...] = jnp.concatenate([p1, p2], axis=1)


def _edge_lin(ewt, w, b):
    eb = 16000
    return pl.pallas_call(
        _edge_body,
        grid=(N_EDGES // eb,),
        in_specs=[
            pl.BlockSpec((DE, eb), lambda i: (0, i)),
            pl.BlockSpec((DE, D), lambda i: (0, 0)),
            pl.BlockSpec((1, D), lambda i: (0, 0)),
        ],
        out_specs=pl.BlockSpec((eb // 2, D), lambda i: (i, 0)),
        out_shape=jax.ShapeDtypeStruct((N_EDGES // 2, D), jnp.uint32),
    )(ewt, w, b)


def _sc_agg_body(h_hbm, src_hbm, dst_hbm, e_hbm, out_hbm,
                 sb0, sb1, db0, db1, hg0, hg1, ev0, ev1, ms, agg_sh,
                 si0, si1, sd0, sd1, sg0, sg1, se0, se1):
    cid = lax.axis_index("c")
    sid = lax.axis_index("s")
    wid = sid * NC + cid
    ebase = wid * EPW

    sb = (sb0, sb1)
    db = (db0, db1)
    hg = (hg0, hg1)
    ev = (ev0, ev1)
    si = (si0, si1)
    sd = (sd0, sd1)
    sg = (sg0, sg1)
    se = (se0, se1)

    zero = jnp.zeros((16,), jnp.float32)

    def zbody(i, _):
        for j in range(D // 16):
            ms[i, pl.ds(j * 16, 16)] = zero
        return 0

    lax.fori_loop(0, C, zbody, 0)
    r0 = sid * ROWS_PER_TILE
    for t in range(ROWS_PER_TILE // C):
        pltpu.sync_copy(ms, agg_sh.at[pl.ds(r0 + t * C, C)])
    plsc.subcore_barrier()

    def issue_src(b, c):
        pltpu.async_copy(src_hbm.at[pl.ds(ebase + c * C, C)], sb[b], si[b])

    def issue_dst(b, c):
        pltpu.async_copy(dst_hbm.at[pl.ds(ebase + c * C, C)], db[b], sd[b])

    def issue_data(b, c):
        pltpu.async_copy(h_hbm.at[sb[b]], hg[b], sg[b])
        pltpu.async_copy(e_hbm.at[pl.ds(ebase + c * C, C)], ev[b], se[b])

    def wait_src(b):
        pltpu.make_async_copy(src_hbm.at[pl.ds(0, C)], sb[b], si[b]).wait()

    def wait_dst(b):
        pltpu.make_async_copy(dst_hbm.at[pl.ds(0, C)], db[b], sd[b]).wait()

    def wait_data(b):
        pltpu.make_async_copy(h_hbm.at[sb[b]], hg[b], sg[b]).wait()
        pltpu.make_async_copy(e_hbm.at[pl.ds(0, C)], ev[b], se[b]).wait()

    hi_mask = jnp.uint32(0xFFFF0000)
    sixteen = jnp.uint32(16)

    def compute(b):
        def cbody(i, _):
            for q in range(D // 32):
                wh = hg[b][i, pl.ds(16 * q, 16)]
                we = ev[b][i, pl.ds(16 * q, 16)]
                u = (lax.bitcast_convert_type(wh << sixteen, jnp.float32)
                     + lax.bitcast_convert_type(we << sixteen, jnp.float32))
                v = (lax.bitcast_convert_type(wh & hi_mask, jnp.float32)
                     + lax.bitcast_convert_type(we & hi_mask, jnp.float32))
                ms[i, pl.ds(16 * q, 16)] = jnp.maximum(u, 0.0)
                ms[i, pl.ds(D // 2 + 16 * q, 16)] = jnp.maximum(v, 0.0)
            return 0
        lax.fori_loop(0, C, cbody, 0)

    issue_src(0, 0)
    issue_dst(0, 0)
    issue_src(1, 1)
    issue_dst(1, 1)
    wait_src(0)
    issue_data(0, 0)

    def body(j, _):
        for b in (0, 1):
            nb = 1 - b
            c = 2 * j + b
            c2 = jnp.minimum(c + 2, NCH - 1)
            wait_src(nb)
            issue_data(nb, c + 1)
            wait_data(b)
            issue_src(b, c2)
            compute(b)
            wait_dst(b)
            pltpu.sync_copy(ms, agg_sh.at[db[b]], add=True)
            issue_dst(b, c2)
        return 0

    lax.fori_loop(0, NPAIR, body, 0)

    wait_data(0)
    compute(0)
    wait_dst(0)
    pltpu.sync_copy(ms, agg_sh.at[db[0]], add=True)
    wait_src(1)
    wait_dst(1)

    plsc.subcore_barrier()

    pltpu.sync_copy(agg_sh.at[pl.ds(r0, ROWS_PER_TILE)],
                    out_hbm.at[cid, pl.ds(r0, ROWS_PER_TILE)])


_sc_agg = pl.kernel(
    _sc_agg_body,
    out_type=jax.ShapeDtypeStruct((NC, N_PAD, D), jnp.float32),
    mesh=plsc.VectorSubcoreMesh(core_axis_name="c", subcore_axis_name="s"),
    compiler_params=pltpu.CompilerParams(use_tc_tiling_on_sc=False),
    scratch_types=[
        pltpu.VMEM((C,), jnp.int32),
        pltpu.VMEM((C,), jnp.int32),
        pltpu.VMEM((C,), jnp.int32),
        pltpu.VMEM((C,), jnp.int32),
        pltpu.VMEM((C, D // 2), jnp.uint32),
        pltpu.VMEM((C, D // 2), jnp.uint32),
        pltpu.VMEM((C, D // 2), jnp.uint32),
        pltpu.VMEM((C, D // 2), jnp.uint32),
        pltpu.VMEM((C, D), jnp.float32),
        pltpu.VMEM_SHARED((N_PAD, D), jnp.float32),
        pltpu.SemaphoreType.DMA,
        pltpu.SemaphoreType.DMA,
        pltpu.SemaphoreType.DMA,
        pltpu.SemaphoreType.DMA,
        pltpu.SemaphoreType.DMA,
        pltpu.SemaphoreType.DMA,
        pltpu.SemaphoreType.DMA,
        pltpu.SemaphoreType.DMA,
    ],
)


def _post_body(agg_ref0, agg_ref1, h_ref, w1t, b1r, w2t, b2r, wpz, wph, bp,
               out_ref):
    h = h_ref[...]
    z = agg_ref0[0] + agg_ref1[0] + h
    z = _leaky(jnp.dot(z, w1t[...], preferred_element_type=jnp.float32) + b1r[...])
    z = jnp.tanh(jnp.dot(z, w2t[...], preferred_element_type=jnp.float32) + b2r[...])
    o = (jnp.dot(z, wpz[...], preferred_element_type=jnp.float32)
         + jnp.dot(h, wph[...], preferred_element_type=jnp.float32) + bp[...])
    out_ref[...] = jnp.tanh(o)


def _post(agg2, h, w1t, b1, w2t, b2, wpz, wph, bp):
    rb = 2000
    mat = pl.BlockSpec((rb, D), lambda i: (i, 0))
    wsp = pl.BlockSpec((D, D), lambda i: (0, 0))
    bsp = pl.BlockSpec((1, D), lambda i: (0, 0))
    a0 = pl.BlockSpec((1, rb, D), lambda i: (0, i, 0))
    a1 = pl.BlockSpec((1, rb, D), lambda i: (1, i, 0))
    return pl.pallas_call(
        _post_body,
        grid=(N_NODES // rb,),
        in_specs=[a0, a1, mat, wsp, bsp, wsp, bsp, wsp, wsp, bsp],
        out_specs=mat,
        out_shape=jax.ShapeDtypeStruct((N_NODES, D), jnp.float32),
    )(agg2, agg2, h, w1t, b1, w2t, b2, wpz, wph, bp)


@jax.jit
def kernel(x, edge_index, edge_weight, W_prep, b_prep, W_e, b_e,
           W1, b1, W2, b2, W_post, b_post):
    h, hp = _prep(x, W_prep.T, b_prep.reshape(1, D))
    e = _edge_lin(edge_weight.T, W_e.T, b_e.reshape(1, D))
    eb = 16000
    src = (edge_index[0].reshape(N_EDGES // eb, 2, eb // 2)
           .swapaxes(1, 2).reshape(-1))
    dst = (edge_index[1].reshape(N_EDGES // eb, 2, eb // 2)
           .swapaxes(1, 2).reshape(-1))
    agg2 = _sc_agg(hp, src, dst, e.reshape(N_EDGES, D // 2))
    return _post(agg2, h,
                 W1.T, b1.reshape(1, D), W2.T, b2.reshape(1, D),
                 W_post[:, :D].T, W_post[:, D:].T, b_post.reshape(1, D))


# speedup vs baseline: 2.2153x; 2.2153x over previous
import jax
import jax.numpy as jnp
from jax import lax
from jax.experimental import pallas as pl
from jax.experimental.pallas import tpu as pltpu
from jax.experimental.pallas import tpu_sc as plsc

N_NODES = 10000
N_EDGES = 320000
D = 128
DE = 16
NEG = 0.01

NC = 2
NS = 16
NW = NC * NS

EPW = N_EDGES // NW
C = 80
NCH = EPW // C
NPAIR = (NCH - 1) // 2

N_PAD = 10240
ROWS_PER_TILE = N_PAD // NS

def _leaky(v):
    return jnp.where(v >= 0, v, NEG * v)


def _prep_body(x_ref, wt_ref, b_ref, h_ref):
    v = jnp.dot(x_ref[...], wt_ref[...], preferred_element_type=jnp.float32)
    h_ref[...] = _leaky(v + b_ref[...])


def _prep(x, wt, b):
    rb = 2000
    return pl.pallas_call(
        _prep_body,
        grid=(N_NODES // rb,),
        in_specs=[
            pl.BlockSpec((rb, D), lambda i: (i, 0)),
            pl.BlockSpec((D, D), lambda i: (0, 0)),
            pl.BlockSpec((1, D), lambda i: (0, 0)),
        ],
        out_specs=pl.BlockSpec((rb, D), lambda i: (i, 0)),
        out_shape=jax.ShapeDtypeStruct((N_NODES, D), jnp.float32),
    )(x, wt, b)


def _edge_body(ewt_ref, w_ref, b_ref, e_ref):
    v = lax.dot_general(ewt_ref[...], w_ref[...],
                        dimension_numbers=(((0,), (0,)), ((), ())),
                        preferred_element_type=jnp.float32)
    v = v + b_ref[...]
    r = lax.bitcast_convert_type(v, jnp.uint32) + jnp.uint32(0x8000)
    e_ref[...] = (r[:, :D // 2] >> jnp.uint32(16)) | (r[:, D // 2:] & jnp.uint32(0xFFFF0000))


def _edge_lin(ewt, w, b):
    eb = 32000
    return pl.pallas_call(
        _edge_body,
        grid=(N_EDGES // eb,),
        in_specs=[
            pl.BlockSpec((DE, eb), lambda i: (0, i)),
            pl.BlockSpec((DE, D), lambda i: (0, 0)),
            pl.BlockSpec((1, D), lambda i: (0, 0)),
        ],
        out_specs=pl.BlockSpec((eb, D // 2), lambda i: (i, 0)),
        out_shape=jax.ShapeDtypeStruct((N_EDGES, D // 2), jnp.uint32),
    )(ewt, w, b)


def _sc_agg_body(h_hbm, src_hbm, dst_hbm, e_hbm, out_hbm,
                 sb0, sb1, db0, db1, hg0, hg1, ev0, ev1, agg_sh,
                 si0, si1, sd0, sd1, sg0, sg1, se0, se1):
    cid = lax.axis_index("c")
    sid = lax.axis_index("s")
    wid = sid * NC + cid
    ebase = wid * EPW

    sb = (sb0, sb1)
    db = (db0, db1)
    hg = (hg0, hg1)
    ev = (ev0, ev1)
    si = (si0, si1)
    sd = (sd0, sd1)
    sg = (sg0, sg1)
    se = (se0, se1)

    zero = jnp.zeros((16,), jnp.float32)

    def zbody(i, _):
        for j in range(D // 16):
            hg0[i, pl.ds(j * 16, 16)] = zero
        return 0

    lax.fori_loop(0, C, zbody, 0)
    r0 = sid * ROWS_PER_TILE
    for t in range(ROWS_PER_TILE // C):
        pltpu.sync_copy(hg0, agg_sh.at[pl.ds(r0 + t * C, C)])
    plsc.subcore_barrier()

    def issue_src(b, c):
        pltpu.async_copy(src_hbm.at[pl.ds(ebase + c * C, C)], sb[b], si[b])

    def issue_dst(b, c):
        pltpu.async_copy(dst_hbm.at[pl.ds(ebase + c * C, C)], db[b], sd[b])

    def issue_data(b, c):
        pltpu.async_copy(h_hbm.at[sb[b]], hg[b], sg[b])
        pltpu.async_copy(e_hbm.at[pl.ds(ebase + c * C, C)], ev[b], se[b])

    def wait_src(b):
        pltpu.make_async_copy(src_hbm.at[pl.ds(0, C)], sb[b], si[b]).wait()

    def wait_dst(b):
        pltpu.make_async_copy(dst_hbm.at[pl.ds(0, C)], db[b], sd[b]).wait()

    def wait_data(b):
        pltpu.make_async_copy(h_hbm.at[sb[b]], hg[b], sg[b]).wait()
        pltpu.make_async_copy(e_hbm.at[pl.ds(0, C)], ev[b], se[b]).wait()

    hi_mask = jnp.uint32(0xFFFF0000)
    sixteen = jnp.uint32(16)

    def compute(b):
        def cbody(i, _):
            for q in range(D // 32):
                w = ev[b][i, pl.ds(16 * q, 16)]
                u = lax.bitcast_convert_type(w << sixteen, jnp.float32)
                v = lax.bitcast_convert_type(w & hi_mask, jnp.float32)
                s0 = pl.ds(16 * q, 16)
                s1 = pl.ds(D // 2 + 16 * q, 16)
                hg[b][i, s0] = jnp.maximum(hg[b][i, s0] + u, 0.0)
                hg[b][i, s1] = jnp.maximum(hg[b][i, s1] + v, 0.0)
            return 0
        lax.fori_loop(0, C, cbody, 0)

    issue_src(0, 0)
    issue_dst(0, 0)
    issue_src(1, 1)
    issue_dst(1, 1)
    wait_src(0)
    issue_data(0, 0)

    def body(j, _):
        for b in (0, 1):
            nb = 1 - b
            c = 2 * j + b
            c2 = jnp.minimum(c + 2, NCH - 1)
            wait_src(nb)
            issue_data(nb, c + 1)
            wait_data(b)
            issue_src(b, c2)
            compute(b)
            wait_dst(b)
            pltpu.sync_copy(hg[b], agg_sh.at[db[b]], add=True)
            issue_dst(b, c2)
        return 0

    lax.fori_loop(0, NPAIR, body, 0)

    wait_data(0)
    compute(0)
    wait_dst(0)
    pltpu.sync_copy(hg[0], agg_sh.at[db[0]], add=True)
    wait_src(1)
    wait_dst(1)

    plsc.subcore_barrier()

    pltpu.sync_copy(agg_sh.at[pl.ds(r0, ROWS_PER_TILE)],
                    out_hbm.at[cid, pl.ds(r0, ROWS_PER_TILE)])


_sc_agg = pl.kernel(
    _sc_agg_body,
    out_type=jax.ShapeDtypeStruct((NC, N_PAD, D), jnp.float32),
    mesh=plsc.VectorSubcoreMesh(core_axis_name="c", subcore_axis_name="s"),
    scratch_types=[
        pltpu.VMEM((C,), jnp.int32),
        pltpu.VMEM((C,), jnp.int32),
        pltpu.VMEM((C,), jnp.int32),
        pltpu.VMEM((C,), jnp.int32),
        pltpu.VMEM((C, D), jnp.float32),
        pltpu.VMEM((C, D), jnp.float32),
        pltpu.VMEM((C, D // 2), jnp.uint32),
        pltpu.VMEM((C, D // 2), jnp.uint32),
        pltpu.VMEM_SHARED((N_PAD, D), jnp.float32),
        pltpu.SemaphoreType.DMA,
        pltpu.SemaphoreType.DMA,
        pltpu.SemaphoreType.DMA,
        pltpu.SemaphoreType.DMA,
        pltpu.SemaphoreType.DMA,
        pltpu.SemaphoreType.DMA,
        pltpu.SemaphoreType.DMA,
        pltpu.SemaphoreType.DMA,
    ],
)


def _post_body(agg_ref0, agg_ref1, h_ref, w1t, b1r, w2t, b2r, wpz, wph, bp,
               out_ref):
    h = h_ref[...]
    z = agg_ref0[0] + agg_ref1[0] + h
    z = _leaky(jnp.dot(z, w1t[...], preferred_element_type=jnp.float32) + b1r[...])
    z = jnp.tanh(jnp.dot(z, w2t[...], preferred_element_type=jnp.float32) + b2r[...])
    o = (jnp.dot(z, wpz[...], preferred_element_type=jnp.float32)
         + jnp.dot(h, wph[...], preferred_element_type=jnp.float32) + bp[...])
    out_ref[...] = jnp.tanh(o)


def _post(agg2, h, w1t, b1, w2t, b2, wpz, wph, bp):
    rb = 2000
    mat = pl.BlockSpec((rb, D), lambda i: (i, 0))
    wsp = pl.BlockSpec((D, D), lambda i: (0, 0))
    bsp = pl.BlockSpec((1, D), lambda i: (0, 0))
    a0 = pl.BlockSpec((1, rb, D), lambda i: (0, i, 0))
    a1 = pl.BlockSpec((1, rb, D), lambda i: (1, i, 0))
    return pl.pallas_call(
        _post_body,
        grid=(N_NODES // rb,),
        in_specs=[a0, a1, mat, wsp, bsp, wsp, bsp, wsp, wsp, bsp],
        out_specs=mat,
        out_shape=jax.ShapeDtypeStruct((N_NODES, D), jnp.float32),
    )(agg2, agg2, h, w1t, b1, w2t, b2, wpz, wph, bp)


@jax.jit
def kernel(x, edge_index, edge_weight, W_prep, b_prep, W_e, b_e,
           W1, b1, W2, b2, W_post, b_post):
    h = _prep(x, W_prep.T, b_prep.reshape(1, D))
    e = _edge_lin(edge_weight.T, W_e.T, b_e.reshape(1, D))
    agg2 = _sc_agg(h, edge_index[0], edge_index[1], e)
    return _post(agg2, h,
                 W1.T, b1.reshape(1, D), W2.T, b2.reshape(1, D),
                 W_post[:, :D].T, W_post[:, D:].T, b_post.reshape(1, D))


# speedup vs baseline: 2.2222x; 1.0031x over previous
import jax
import jax.numpy as jnp
from jax import lax
from jax.experimental import pallas as pl
from jax.experimental.pallas import tpu as pltpu
from jax.experimental.pallas import tpu_sc as plsc

N_NODES = 10000
N_EDGES = 320000
D = 128
DE = 16
NEG = 0.01

NC = 2
NS = 16
NW = NC * NS

EPW = N_EDGES // NW
C = 80
NCH = EPW // C
NPAIR = (NCH - 1) // 2

N_PAD = 10240
ROWS_PER_TILE = N_PAD // NS

def _leaky(v):
    return jnp.where(v >= 0, v, NEG * v)


def _prep_body(x_ref, wt_ref, b_ref, h_ref):
    v = jnp.dot(x_ref[...], wt_ref[...], preferred_element_type=jnp.float32)
    h_ref[...] = _leaky(v + b_ref[...])


def _prep(x, wt, b):
    rb = 2000
    return pl.pallas_call(
        _prep_body,
        grid=(N_NODES // rb,),
        in_specs=[
            pl.BlockSpec((rb, D), lambda i: (i, 0)),
            pl.BlockSpec((D, D), lambda i: (0, 0)),
            pl.BlockSpec((1, D), lambda i: (0, 0)),
        ],
        out_specs=pl.BlockSpec((rb, D), lambda i: (i, 0)),
        out_shape=jax.ShapeDtypeStruct((N_NODES, D), jnp.float32),
    )(x, wt, b)


def _edge_body(ewt_ref, w_ref, b_ref, e_ref):
    v = lax.dot_general(ewt_ref[...], w_ref[...],
                        dimension_numbers=(((0,), (0,)), ((), ())),
                        preferred_element_type=jnp.float32)
    v = v + b_ref[...]
    r = lax.bitcast_convert_type(v, jnp.uint32) + jnp.uint32(0x8000)
    e_ref[...] = (r[:, :D // 2] >> jnp.uint32(16)) | (r[:, D // 2:] & jnp.uint32(0xFFFF0000))


def _edge_lin(ewt, w, b):
    eb = 16000
    return pl.pallas_call(
        _edge_body,
        grid=(N_EDGES // eb,),
        in_specs=[
            pl.BlockSpec((DE, eb), lambda i: (0, i)),
            pl.BlockSpec((DE, D), lambda i: (0, 0)),
            pl.BlockSpec((1, D), lambda i: (0, 0)),
        ],
        out_specs=pl.BlockSpec((eb, D // 2), lambda i: (i, 0)),
        out_shape=jax.ShapeDtypeStruct((N_EDGES, D // 2), jnp.uint32),
    )(ewt, w, b)


def _sc_agg_body(h_hbm, src_hbm, dst_hbm, e_hbm, out_hbm,
                 sb0, sb1, db0, db1, hg0, hg1, ev0, ev1, agg_sh,
                 si0, si1, sd0, sd1, sg0, sg1, se0, se1):
    cid = lax.axis_index("c")
    sid = lax.axis_index("s")
    wid = sid * NC + cid
    ebase = wid * EPW

    sb = (sb0, sb1)
    db = (db0, db1)
    hg = (hg0, hg1)
    ev = (ev0, ev1)
    si = (si0, si1)
    sd = (sd0, sd1)
    sg = (sg0, sg1)
    se = (se0, se1)

    zero = jnp.zeros((16,), jnp.float32)

    def zbody(i, _):
        for j in range(D // 16):
            hg0[i, pl.ds(j * 16, 16)] = zero
        return 0

    lax.fori_loop(0, C, zbody, 0)
    r0 = sid * ROWS_PER_TILE
    for t in range(ROWS_PER_TILE // C):
        pltpu.sync_copy(hg0, agg_sh.at[pl.ds(r0 + t * C, C)])
    plsc.subcore_barrier()

    def issue_src(b, c):
        pltpu.async_copy(src_hbm.at[pl.ds(ebase + c * C, C)], sb[b], si[b])

    def issue_dst(b, c):
        pltpu.async_copy(dst_hbm.at[pl.ds(ebase + c * C, C)], db[b], sd[b])

    def issue_data(b, c):
        pltpu.async_copy(h_hbm.at[sb[b]], hg[b], sg[b])
        pltpu.async_copy(e_hbm.at[pl.ds(ebase + c * C, C)], ev[b], se[b])

    def wait_src(b):
        pltpu.make_async_copy(src_hbm.at[pl.ds(0, C)], sb[b], si[b]).wait()

    def wait_dst(b):
        pltpu.make_async_copy(dst_hbm.at[pl.ds(0, C)], db[b], sd[b]).wait()

    def wait_data(b):
        pltpu.make_async_copy(h_hbm.at[sb[b]], hg[b], sg[b]).wait()
        pltpu.make_async_copy(e_hbm.at[pl.ds(0, C)], ev[b], se[b]).wait()

    hi_mask = jnp.uint32(0xFFFF0000)
    sixteen = jnp.uint32(16)

    def compute(b):
        def cbody(i, _):
            for q in range(D // 32):
                w = ev[b][i, pl.ds(16 * q, 16)]
                u = lax.bitcast_convert_type(w << sixteen, jnp.float32)
                v = lax.bitcast_convert_type(w & hi_mask, jnp.float32)
                s0 = pl.ds(16 * q, 16)
                s1 = pl.ds(D // 2 + 16 * q, 16)
                hg[b][i, s0] = jnp.maximum(hg[b][i, s0] + u, 0.0)
                hg[b][i, s1] = jnp.maximum(hg[b][i, s1] + v, 0.0)
            return 0
        lax.fori_loop(0, C, cbody, 0)

    issue_src(0, 0)
    issue_dst(0, 0)
    issue_src(1, 1)
    issue_dst(1, 1)
    wait_src(0)
    issue_data(0, 0)

    def body(j, _):
        for b in (0, 1):
            nb = 1 - b
            c = 2 * j + b
            c2 = jnp.minimum(c + 2, NCH - 1)
            wait_src(nb)
            issue_data(nb, c + 1)
            wait_data(b)
            issue_src(b, c2)
            compute(b)
            wait_dst(b)
            pltpu.sync_copy(hg[b], agg_sh.at[db[b]], add=True)
            issue_dst(b, c2)
        return 0

    lax.fori_loop(0, NPAIR, body, 0)

    wait_data(0)
    compute(0)
    wait_dst(0)
    pltpu.sync_copy(hg[0], agg_sh.at[db[0]], add=True)
    wait_src(1)
    wait_dst(1)

    plsc.subcore_barrier()

    pltpu.sync_copy(agg_sh.at[pl.ds(r0, ROWS_PER_TILE)],
                    out_hbm.at[cid, pl.ds(r0, ROWS_PER_TILE)])


_sc_agg = pl.kernel(
    _sc_agg_body,
    out_type=jax.ShapeDtypeStruct((NC, N_PAD, D), jnp.float32),
    mesh=plsc.VectorSubcoreMesh(core_axis_name="c", subcore_axis_name="s"),
    scratch_types=[
        pltpu.VMEM((C,), jnp.int32),
        pltpu.VMEM((C,), jnp.int32),
        pltpu.VMEM((C,), jnp.int32),
        pltpu.VMEM((C,), jnp.int32),
        pltpu.VMEM((C, D), jnp.float32),
        pltpu.VMEM((C, D), jnp.float32),
        pltpu.VMEM((C, D // 2), jnp.uint32),
        pltpu.VMEM((C, D // 2), jnp.uint32),
        pltpu.VMEM_SHARED((N_PAD, D), jnp.float32),
        pltpu.SemaphoreType.DMA,
        pltpu.SemaphoreType.DMA,
        pltpu.SemaphoreType.DMA,
        pltpu.SemaphoreType.DMA,
        pltpu.SemaphoreType.DMA,
        pltpu.SemaphoreType.DMA,
        pltpu.SemaphoreType.DMA,
        pltpu.SemaphoreType.DMA,
    ],
)


def _post_body(agg_ref0, agg_ref1, h_ref, w1t, b1r, w2t, b2r, wpz, wph, bp,
               out_ref):
    h = h_ref[...]
    z = agg_ref0[0] + agg_ref1[0] + h
    z = _leaky(jnp.dot(z, w1t[...], preferred_element_type=jnp.float32) + b1r[...])
    z = jnp.tanh(jnp.dot(z, w2t[...], preferred_element_type=jnp.float32) + b2r[...])
    o = (jnp.dot(z, wpz[...], preferred_element_type=jnp.float32)
         + jnp.dot(h, wph[...], preferred_element_type=jnp.float32) + bp[...])
    out_ref[...] = jnp.tanh(o)


def _post(agg2, h, w1t, b1, w2t, b2, wpz, wph, bp):
    rb = 2000
    mat = pl.BlockSpec((rb, D), lambda i: (i, 0))
    wsp = pl.BlockSpec((D, D), lambda i: (0, 0))
    bsp = pl.BlockSpec((1, D), lambda i: (0, 0))
    a0 = pl.BlockSpec((1, rb, D), lambda i: (0, i, 0))
    a1 = pl.BlockSpec((1, rb, D), lambda i: (1, i, 0))
    return pl.pallas_call(
        _post_body,
        grid=(N_NODES // rb,),
        in_specs=[a0, a1, mat, wsp, bsp, wsp, bsp, wsp, wsp, bsp],
        out_specs=mat,
        out_shape=jax.ShapeDtypeStruct((N_NODES, D), jnp.float32),
    )(agg2, agg2, h, w1t, b1, w2t, b2, wpz, wph, bp)


@jax.jit
def kernel(x, edge_index, edge_weight, W_prep, b_prep, W_e, b_e,
           W1, b1, W2, b2, W_post, b_post):
    h = _prep(x, W_prep.T, b_prep.reshape(1, D))
    e = _edge_lin(edge_weight.T, W_e.T, b_e.reshape(1, D))
    agg2 = _sc_agg(h, edge_index[0], edge_index[1], e)
    return _post(agg2, h,
                 W1.T, b1.reshape(1, D), W2.T, b2.reshape(1, D),
                 W_post[:, :D].T, W_post[:, D:].T, b_post.reshape(1, D))


# speedup vs baseline: 2.2248x; 1.0012x over previous
import jax
import jax.numpy as jnp
from jax import lax
from jax.experimental import pallas as pl
from jax.experimental.pallas import tpu as pltpu
from jax.experimental.pallas import tpu_sc as plsc

N_NODES = 10000
N_EDGES = 320000
D = 128
DE = 16
NEG = 0.01

NC = 2
NS = 16
NW = NC * NS

EPW = N_EDGES // NW
C = 80
NCH = EPW // C
NPAIR = (NCH - 1) // 2

N_PAD = 10240
ROWS_PER_TILE = N_PAD // NS

def _leaky(v):
    return jnp.where(v >= 0, v, NEG * v)


def _prep_body(x_ref, wt_ref, b_ref, h_ref):
    v = jnp.dot(x_ref[...], wt_ref[...], preferred_element_type=jnp.float32)
    h_ref[...] = _leaky(v + b_ref[...])


def _prep(x, wt, b):
    rb = 2000
    return pl.pallas_call(
        _prep_body,
        grid=(N_NODES // rb,),
        in_specs=[
            pl.BlockSpec((rb, D), lambda i: (i, 0)),
            pl.BlockSpec((D, D), lambda i: (0, 0)),
            pl.BlockSpec((1, D), lambda i: (0, 0)),
        ],
        out_specs=pl.BlockSpec((rb, D), lambda i: (i, 0)),
        out_shape=jax.ShapeDtypeStruct((N_NODES, D), jnp.float32),
    )(x, wt, b)


def _edge_body(ewt_ref, w_ref, b_ref, e_ref):
    v = lax.dot_general(ewt_ref[...], w_ref[...],
                        dimension_numbers=(((0,), (0,)), ((), ())),
                        preferred_element_type=jnp.float32)
    v = v + b_ref[...]
    r = lax.bitcast_convert_type(v, jnp.uint32) + jnp.uint32(0x8000)
    e_ref[...] = (r[:, :D // 2] >> jnp.uint32(16)) | (r[:, D // 2:] & jnp.uint32(0xFFFF0000))


def _edge_lin(ewt, w, b):
    eb = 16000
    return pl.pallas_call(
        _edge_body,
        grid=(N_EDGES // eb,),
        in_specs=[
            pl.BlockSpec((DE, eb), lambda i: (0, i)),
            pl.BlockSpec((DE, D), lambda i: (0, 0)),
            pl.BlockSpec((1, D), lambda i: (0, 0)),
        ],
        out_specs=pl.BlockSpec((eb, D // 2), lambda i: (i, 0)),
        out_shape=jax.ShapeDtypeStruct((N_EDGES, D // 2), jnp.uint32),
    )(ewt, w, b)


def _sc_agg_body(h_hbm, src_hbm, dst_hbm, e_hbm, out_hbm,
                 sb0, sb1, db0, db1, hg0, hg1, ev0, ev1, agg_sh,
                 si0, si1, sd0, sd1, sg0, sg1, se0, se1, ss0, ss1):
    cid = lax.axis_index("c")
    sid = lax.axis_index("s")
    wid = sid * NC + cid
    ebase = wid * EPW

    sb = (sb0, sb1)
    db = (db0, db1)
    hg = (hg0, hg1)
    ev = (ev0, ev1)
    si = (si0, si1)
    sd = (sd0, sd1)
    sg = (sg0, sg1)
    se = (se0, se1)
    ss = (ss0, ss1)

    zero = jnp.zeros((16,), jnp.float32)

    def zbody(i, _):
        for j in range(D // 16):
            hg0[i, pl.ds(j * 16, 16)] = zero
        return 0

    lax.fori_loop(0, C, zbody, 0)
    r0 = sid * ROWS_PER_TILE
    for t in range(ROWS_PER_TILE // C):
        pltpu.sync_copy(hg0, agg_sh.at[pl.ds(r0 + t * C, C)])
    plsc.subcore_barrier()

    def issue_src(b, c):
        pltpu.async_copy(src_hbm.at[pl.ds(ebase + c * C, C)], sb[b], si[b])

    def issue_dst(b, c):
        pltpu.async_copy(dst_hbm.at[pl.ds(ebase + c * C, C)], db[b], sd[b])

    def issue_data(b, c):
        pltpu.async_copy(h_hbm.at[sb[b]], hg[b], sg[b])
        pltpu.async_copy(e_hbm.at[pl.ds(ebase + c * C, C)], ev[b], se[b])

    def wait_src(b):
        pltpu.make_async_copy(src_hbm.at[pl.ds(0, C)], sb[b], si[b]).wait()

    def wait_dst(b):
        pltpu.make_async_copy(dst_hbm.at[pl.ds(0, C)], db[b], sd[b]).wait()

    def wait_data(b):
        pltpu.make_async_copy(h_hbm.at[sb[b]], hg[b], sg[b]).wait()
        pltpu.make_async_copy(e_hbm.at[pl.ds(0, C)], ev[b], se[b]).wait()

    hi_mask = jnp.uint32(0xFFFF0000)
    sixteen = jnp.uint32(16)

    def compute(b):
        def cbody(i, _):
            for q in range(D // 32):
                w = ev[b][i, pl.ds(16 * q, 16)]
                u = lax.bitcast_convert_type(w << sixteen, jnp.float32)
                v = lax.bitcast_convert_type(w & hi_mask, jnp.float32)
                s0 = pl.ds(16 * q, 16)
                s1 = pl.ds(D // 2 + 16 * q, 16)
                hg[b][i, s0] = jnp.maximum(hg[b][i, s0] + u, 0.0)
                hg[b][i, s1] = jnp.maximum(hg[b][i, s1] + v, 0.0)
            return 0
        lax.fori_loop(0, C, cbody, 0)

    issue_src(0, 0)
    issue_dst(0, 0)
    issue_src(1, 1)
    issue_dst(1, 1)
    wait_src(0)
    issue_data(0, 0)

    def wait_scatter(b):
        pltpu.make_async_copy(hg[b], agg_sh.at[db[b]], ss[b]).wait()

    def body(j, _):
        for b in (0, 1):
            nb = 1 - b
            c = 2 * j + b
            c2 = jnp.minimum(c + 2, NCH - 1)
            wait_src(nb)
            if b == 0:
                @pl.when(j > 0)
                def _():
                    wait_scatter(nb)
            else:
                wait_scatter(nb)
            issue_data(nb, c + 1)
            wait_data(b)
            issue_src(b, c2)
            compute(b)
            wait_dst(b)
            pltpu.async_copy(hg[b], agg_sh.at[db[b]], ss[b], add=True)
            issue_dst(b, c2)
        return 0

    lax.fori_loop(0, NPAIR, body, 0)

    wait_data(0)
    compute(0)
    wait_dst(0)
    pltpu.async_copy(hg[0], agg_sh.at[db[0]], ss[0], add=True)
    wait_scatter(1)
    wait_scatter(0)
    wait_src(1)
    wait_dst(1)

    plsc.subcore_barrier()

    pltpu.sync_copy(agg_sh.at[pl.ds(r0, ROWS_PER_TILE)],
                    out_hbm.at[cid, pl.ds(r0, ROWS_PER_TILE)])


_sc_agg = pl.kernel(
    _sc_agg_body,
    out_type=jax.ShapeDtypeStruct((NC, N_PAD, D), jnp.float32),
    mesh=plsc.VectorSubcoreMesh(core_axis_name="c", subcore_axis_name="s"),
    scratch_types=[
        pltpu.VMEM((C,), jnp.int32),
        pltpu.VMEM((C,), jnp.int32),
        pltpu.VMEM((C,), jnp.int32),
        pltpu.VMEM((C,), jnp.int32),
        pltpu.VMEM((C, D), jnp.float32),
        pltpu.VMEM((C, D), jnp.float32),
        pltpu.VMEM((C, D // 2), jnp.uint32),
        pltpu.VMEM((C, D // 2), jnp.uint32),
        pltpu.VMEM_SHARED((N_PAD, D), jnp.float32),
        pltpu.SemaphoreType.DMA,
        pltpu.SemaphoreType.DMA,
        pltpu.SemaphoreType.DMA,
        pltpu.SemaphoreType.DMA,
        pltpu.SemaphoreType.DMA,
        pltpu.SemaphoreType.DMA,
        pltpu.SemaphoreType.DMA,
        pltpu.SemaphoreType.DMA,
        pltpu.SemaphoreType.DMA,
        pltpu.SemaphoreType.DMA,
    ],
)


def _post_body(agg_ref0, agg_ref1, h_ref, w1t, b1r, w2t, b2r, wpz, wph, bp,
               out_ref):
    h = h_ref[...]
    z = agg_ref0[0] + agg_ref1[0] + h
    z = _leaky(jnp.dot(z, w1t[...], preferred_element_type=jnp.float32) + b1r[...])
    z = jnp.tanh(jnp.dot(z, w2t[...], preferred_element_type=jnp.float32) + b2r[...])
    o = (jnp.dot(z, wpz[...], preferred_element_type=jnp.float32)
         + jnp.dot(h, wph[...], preferred_element_type=jnp.float32) + bp[...])
    out_ref[...] = jnp.tanh(o)


def _post(agg2, h, w1t, b1, w2t, b2, wpz, wph, bp):
    rb = 2000
    mat = pl.BlockSpec((rb, D), lambda i: (i, 0))
    wsp = pl.BlockSpec((D, D), lambda i: (0, 0))
    bsp = pl.BlockSpec((1, D), lambda i: (0, 0))
    a0 = pl.BlockSpec((1, rb, D), lambda i: (0, i, 0))
    a1 = pl.BlockSpec((1, rb, D), lambda i: (1, i, 0))
    return pl.pallas_call(
        _post_body,
        grid=(N_NODES // rb,),
        in_specs=[a0, a1, mat, wsp, bsp, wsp, bsp, wsp, wsp, bsp],
        out_specs=mat,
        out_shape=jax.ShapeDtypeStruct((N_NODES, D), jnp.float32),
    )(agg2, agg2, h, w1t, b1, w2t, b2, wpz, wph, bp)


@jax.jit
def kernel(x, edge_index, edge_weight, W_prep, b_prep, W_e, b_e,
           W1, b1, W2, b2, W_post, b_post):
    h = _prep(x, W_prep.T, b_prep.reshape(1, D))
    e = _edge_lin(edge_weight.T, W_e.T, b_e.reshape(1, D))
    agg2 = _sc_agg(h, edge_index[0], edge_index[1], e)
    return _post(agg2, h,
                 W1.T, b1.reshape(1, D), W2.T, b2.reshape(1, D),
                 W_post[:, :D].T, W_post[:, D:].T, b_post.reshape(1, D))
